# baseline (speedup 1.0000x reference)
import functools

import jax
import jax.numpy as jnp
from jax import lax
from jax.experimental import pallas as pl
from jax.experimental.pallas import tpu as pltpu
from jax.experimental.pallas import tpu_sc as plsc

N = 10000
N_EXT = 10240
E = 320000
NC, NS = 2, 16
NW = NC * NS
B = 128
NCH = 80
EPW = NCH * B
E_PAD = NW * EPW
RPW = N_EXT // NS
DUMMY = N
PK = 4
RD = 4


def _tc_first_body(x_ref, w_ref, asr_ref, adr_ref,
                   h_ref, a1_ref, a2_ref, m_ref, msc):
    i = pl.program_id(0)
    h = jnp.dot(x_ref[...], w_ref[...], preferred_element_type=jnp.float32)
    h_ref[...] = h
    a1 = jnp.dot(h, asr_ref[...], preferred_element_type=jnp.float32)
    a2 = jnp.dot(h, adr_ref[...], preferred_element_type=jnp.float32)
    a1_ref[...] = a1
    a2_ref[...] = a2
    bm1 = jnp.max(a1)
    bm2 = jnp.max(a2)

    @pl.when(i == 0)
    def _():
        msc[0] = bm1
        msc[1] = bm2

    @pl.when(i > 0)
    def _():
        msc[0] = jnp.maximum(msc[0], bm1)
        msc[1] = jnp.maximum(msc[1], bm2)

    m_ref[...] = jnp.maximum(msc[0] + msc[1], 0.0).reshape(1, 1)


def _tc_mid_body(p0_ref, p1_ref, s0_ref, s1_ref, bp_ref, w_ref, asr_ref,
                 adr_ref, h_ref, a1_ref, a2_ref, m_ref, msc):
    i = pl.program_id(0)
    s = s0_ref[...] + s1_ref[...] + 1e-16
    X = (p0_ref[...] + p1_ref[...]) / s + bp_ref[...]
    X = jnp.maximum(X, 0.0)
    h = jnp.dot(X, w_ref[...], preferred_element_type=jnp.float32)
    h_ref[...] = h
    a1 = jnp.dot(h, asr_ref[...], preferred_element_type=jnp.float32)
    a2 = jnp.dot(h, adr_ref[...], preferred_element_type=jnp.float32)
    a1_ref[...] = a1
    a2_ref[...] = a2
    bm1 = jnp.max(a1)
    bm2 = jnp.max(a2)

    @pl.when(i == 0)
    def _():
        msc[0] = bm1
        msc[1] = bm2

    @pl.when(i > 0)
    def _():
        msc[0] = jnp.maximum(msc[0], bm1)
        msc[1] = jnp.maximum(msc[1], bm2)

    m_ref[...] = jnp.maximum(msc[0] + msc[1], 0.0).reshape(1, 1)


def _tc_layer(X_or_parts, W, a_src, a_dst, first):
    d_in, d = W.shape
    BN = 1280
    grid = (N_EXT // BN,)
    out_shape = (
        jax.ShapeDtypeStruct((N_EXT, d), jnp.float32),
        jax.ShapeDtypeStruct((N_EXT, 1), jnp.float32),
        jax.ShapeDtypeStruct((N_EXT, 1), jnp.float32),
        jax.ShapeDtypeStruct((1, 1), jnp.float32),
    )
    out_specs = (
        pl.BlockSpec((BN, d), lambda i: (i, 0)),
        pl.BlockSpec((BN, 1), lambda i: (i, 0)),
        pl.BlockSpec((BN, 1), lambda i: (i, 0)),
        pl.BlockSpec((1, 1), lambda i: (0, 0)),
    )
    asr = a_src.reshape(d, 1)
    adr = a_dst.reshape(d, 1)
    if first:
        x = X_or_parts
        return pl.pallas_call(
            _tc_first_body,
            grid=grid,
            in_specs=[
                pl.BlockSpec((BN, d_in), lambda i: (i, 0)),
                pl.BlockSpec((d_in, d), lambda i: (0, 0)),
                pl.BlockSpec((d, 1), lambda i: (0, 0)),
                pl.BlockSpec((d, 1), lambda i: (0, 0)),
            ],
            out_specs=out_specs,
            out_shape=out_shape,
            scratch_shapes=[pltpu.SMEM((2,), jnp.float32)],
        )(x, W, asr, adr)
    p0, p1, s0, s1, bp = X_or_parts
    return pl.pallas_call(
        _tc_mid_body,
        grid=grid,
        in_specs=[
            pl.BlockSpec((BN, d_in), lambda i: (i, 0)),
            pl.BlockSpec((BN, d_in), lambda i: (i, 0)),
            pl.BlockSpec((BN, 1), lambda i: (i, 0)),
            pl.BlockSpec((BN, 1), lambda i: (i, 0)),
            pl.BlockSpec((1, d_in), lambda i: (0, 0)),
            pl.BlockSpec((d_in, d), lambda i: (0, 0)),
            pl.BlockSpec((d, 1), lambda i: (0, 0)),
            pl.BlockSpec((d, 1), lambda i: (0, 0)),
        ],
        out_specs=out_specs,
        out_shape=out_shape,
        scratch_shapes=[pltpu.SMEM((2,), jnp.float32)],
    )(p0, p1, s0.reshape(N_EXT, 1), s1.reshape(N_EXT, 1),
      bp.reshape(1, d_in), W, asr, adr)


def _tc_norm_body(p0_ref, p1_ref, s0_ref, s1_ref, b_ref, o_ref):
    s = s0_ref[...] + s1_ref[...] + 1e-16
    o_ref[...] = (p0_ref[...] + p1_ref[...]) / s + b_ref[...]


def _tc_norm(p0, p1, s0, s1, b):
    d = p0.shape[-1]
    BN = 1280
    return pl.pallas_call(
        _tc_norm_body,
        grid=(N_EXT // BN,),
        in_specs=[
            pl.BlockSpec((BN, d), lambda i: (i, 0)),
            pl.BlockSpec((BN, d), lambda i: (i, 0)),
            pl.BlockSpec((BN, 1), lambda i: (i, 0)),
            pl.BlockSpec((BN, 1), lambda i: (i, 0)),
            pl.BlockSpec((1, d), lambda i: (0, 0)),
        ],
        out_specs=pl.BlockSpec((BN, d), lambda i: (i, 0)),
        out_shape=jax.ShapeDtypeStruct((N_EXT, d), jnp.float32),
    )(p0, p1, s0.reshape(N_EXT, 1), s1.reshape(N_EXT, 1), b.reshape(1, d))


def _bcast_lane(vec, l):
    idx = jnp.full((16, 1), l, jnp.int32)
    return lax.gather(
        vec, idx,
        lax.GatherDimensionNumbers(
            offset_dims=(), collapsed_slice_dims=(0,), start_index_map=(0,)),
        slice_sizes=(1,),
        mode=lax.GatherScatterMode.PROMISE_IN_BOUNDS)

@functools.partial(jax.jit, static_argnames=("d",))
def _sc_edge(h, asv, adv, mvec, srcp, dstp, znd, zn, d):
    mesh = plsc.VectorSubcoreMesh(core_axis_name="c", subcore_axis_name="s")

    @functools.partial(
        pl.kernel,
        out_type=(
            jax.ShapeDtypeStruct((NC, N_EXT, d), jnp.float32),
            jax.ShapeDtypeStruct((NC, N_EXT), jnp.float32),
        ),
        mesh=mesh,
        scratch_types=[
            pltpu.VMEM((PK, B), jnp.int32),
            pltpu.VMEM((PK, B), jnp.int32),
            pltpu.VMEM((PK * B,), jnp.float32),
            pltpu.VMEM((PK * B,), jnp.float32),
            pltpu.VMEM((NCH * B,), jnp.float32),
            pltpu.VMEM((16,), jnp.float32),
            pltpu.VMEM((RD, B), jnp.int32),
            pltpu.VMEM((RD, B), jnp.int32),
            pltpu.VMEM((B, d), jnp.float32),
            pltpu.VMEM((B, d), jnp.float32),
            pltpu.VMEM_SHARED((N_EXT, d), jnp.float32),
            pltpu.VMEM_SHARED((N_EXT,), jnp.float32),
            pltpu.SemaphoreType.DMA,
            pltpu.SemaphoreType.DMA,
            pltpu.SemaphoreType.DMA,
            pltpu.SemaphoreType.DMA,
        ],
        compiler_params=pltpu.CompilerParams(use_tc_tiling_on_sc=False),
    )
    def k(h_hbm, as_hbm, ad_hbm, m_hbm, src_hbm, dst_hbm, znd_hbm, zn_hbm,
          acc_out, s_out, sidx1, didx1, ag_v, bg_v, ex_v, m_v,
          sidx2, didx2, rowsA, rowsB, acc_sh, s_sh,
          psem, isem, gA, gB):
        rows = (rowsA, rowsB)
        gsem = (gA, gB)
        cidx = lax.axis_index("c")
        sidx = lax.axis_index("s")
        wid = sidx * NC + cidx
        r0 = sidx * RPW

        pltpu.sync_copy(znd_hbm.at[pl.ds(r0, RPW)], acc_sh.at[pl.ds(r0, RPW)])

        @pl.when(sidx == 0)
        def _():
            pltpu.sync_copy(zn_hbm, s_sh)

        pltpu.sync_copy(m_hbm, m_v)
        plsc.subcore_barrier()

        mv = m_v[...]

        def p1(grp, carry):
            ch0 = grp * PK
            ci = pltpu.async_copy(src_hbm.at[wid, pl.ds(ch0, PK)], sidx1, isem)
            cj = pltpu.async_copy(dst_hbm.at[wid, pl.ds(ch0, PK)], didx1, isem)
            ci.wait()
            cj.wait()
            cps = []
            for kk in range(PK):
                cps.append(pltpu.async_copy(
                    as_hbm.at[sidx1.at[kk]], ag_v.at[pl.ds(kk * B, B)], psem))
                cps.append(pltpu.async_copy(
                    ad_hbm.at[didx1.at[kk]], bg_v.at[pl.ds(kk * B, B)], psem))
            for cp in cps:
                cp.wait()
            for kk in range(PK):
                for j in range(B // 16):
                    sl = pl.ds(kk * B + j * 16, 16)
                    e = ag_v[sl] + bg_v[sl]
                    e = jnp.where(e > 0.0, e, e * 0.2)
                    ex_v[pl.ds((ch0 + kk) * B + j * 16, 16)] = jnp.exp(e - mv)
            for kk in range(PK):
                pltpu.sync_copy(ex_v.at[pl.ds((ch0 + kk) * B, B)],
                                s_sh.at[didx1.at[kk]], add=True)
            return carry

        lax.fori_loop(0, NCH // PK, p1, 0)

        def _scale(buf, ch):
            for g in range(B // 16):
                exg = ex_v[pl.ds(ch * B + g * 16, 16)]
                for l in range(16):
                    j = g * 16 + l
                    exj = _bcast_lane(exg, l)
                    for f in range(d // 16):
                        slf = pl.ds(f * 16, 16)
                        buf[j, slf] = buf[j, slf] * exj

        def _idx_pair(chn, slot):
            pltpu.async_copy(
                src_hbm.at[wid, pl.ds(chn, 1)], sidx2.at[pl.ds(slot, 1)], isem)
            pltpu.async_copy(
                dst_hbm.at[wid, pl.ds(chn, 1)], didx2.at[pl.ds(slot, 1)], isem)

        def _idx_wait():
            pltpu.make_async_copy(
                src_hbm.at[wid, pl.ds(0, 1)], sidx2.at[pl.ds(0, 1)],
                isem).wait()
            pltpu.make_async_copy(
                dst_hbm.at[wid, pl.ds(0, 1)], didx2.at[pl.ds(0, 1)],
                isem).wait()

        _idx_pair(0, 0)
        _idx_wait()
        pltpu.async_copy(h_hbm.at[sidx2.at[0]], rows[0], gsem[0])
        _idx_pair(1, 1)

        def p2(qq, carry):
            for kq in range(4):
                ch = qq * 4 + kq
                nb = (kq + 1) % 2

                @pl.when(ch + 1 < NCH)
                def _():
                    _idx_wait()
                    pltpu.async_copy(
                        h_hbm.at[sidx2.at[(kq + 1) % RD]], rows[nb], gsem[nb])

                @pl.when(ch + 2 < NCH)
                def _():
                    _idx_pair(ch + 2, (kq + 2) % RD)

                pltpu.make_async_copy(
                    h_hbm.at[sidx2.at[0]], rows[kq % 2], gsem[kq % 2]).wait()
                _scale(rows[kq % 2], ch)
                pltpu.sync_copy(rows[kq % 2],
                                acc_sh.at[didx2.at[kq]], add=True)
            return carry

        lax.fori_loop(0, NCH // 4, p2, 0)

        plsc.subcore_barrier()

        pltpu.sync_copy(acc_sh.at[pl.ds(r0, RPW)],
                        acc_out.at[cidx, pl.ds(r0, RPW)])

        @pl.when(sidx == 0)
        def _():
            pltpu.sync_copy(s_sh, s_out.at[cidx])

    return k(h, asv, adv, mvec, srcp, dstp, znd, zn)


def kernel(x, edge_index, W1, a_src1, a_dst1, b1, W2, a_src2, a_dst2, b2,
           W3, a_src3, a_dst3, b3):
    src = edge_index[0].astype(jnp.int32)
    dst = edge_index[1].astype(jnp.int32)
    pad = E_PAD - E
    srcp = jnp.concatenate(
        [src, jnp.full((pad,), DUMMY, jnp.int32)]).reshape(NW, NCH, B)
    dstp = jnp.concatenate(
        [dst, jnp.full((pad,), DUMMY, jnp.int32)]).reshape(NW, NCH, B)
    x_ext = jnp.pad(x, ((0, N_EXT - N), (0, 0)))

    znd128 = jnp.zeros((N_EXT, 128), jnp.float32)
    znd64 = jnp.zeros((N_EXT, 64), jnp.float32)
    zn = jnp.zeros((N_EXT,), jnp.float32)

    h, a1, a2, m = _tc_layer(x_ext, W1, a_src1, a_dst1, first=True)
    mv = jnp.full((16,), m[0, 0], jnp.float32)
    acc, s = _sc_edge(h, a1.reshape(N_EXT), a2.reshape(N_EXT), mv,
                      srcp, dstp, znd128, zn, d=128)

    h, a1, a2, m = _tc_layer(
        (acc[0], acc[1], s[0], s[1], b1), W2, a_src2, a_dst2, first=False)
    mv = jnp.full((16,), m[0, 0], jnp.float32)
    acc, s = _sc_edge(h, a1.reshape(N_EXT), a2.reshape(N_EXT), mv,
                      srcp, dstp, znd64, zn, d=64)

    h, a1, a2, m = _tc_layer(
        (acc[0], acc[1], s[0], s[1], b2), W3, a_src3, a_dst3, first=False)
    mv = jnp.full((16,), m[0, 0], jnp.float32)
    acc, s = _sc_edge(h, a1.reshape(N_EXT), a2.reshape(N_EXT), mv,
                      srcp, dstp, znd64, zn, d=64)

    out = _tc_norm(acc[0], acc[1], s[0], s[1], b3)
    return out[:N]


# speedup vs baseline: 1.8150x; 1.8150x over previous
import functools

import jax
import jax.numpy as jnp
from jax import lax
from jax.experimental import pallas as pl
from jax.experimental.pallas import tpu as pltpu
from jax.experimental.pallas import tpu_sc as plsc

N = 10000
N_EXT = 10240
E = 320000
NC, NS = 2, 16
NW = NC * NS
B = 128
NCH = 80
EPW = NCH * B
E_PAD = NW * EPW
RPW = N_EXT // NS
DUMMY = N
PK = 4
RD = 4


def _tc_first_body(x_ref, w_ref, asr_ref, adr_ref,
                   h_ref, a1_ref, a2_ref, m_ref, msc):
    i = pl.program_id(0)
    h = jnp.dot(x_ref[...], w_ref[...], preferred_element_type=jnp.float32)
    h_ref[...] = h
    a1 = jnp.dot(h, asr_ref[...], preferred_element_type=jnp.float32)
    a2 = jnp.dot(h, adr_ref[...], preferred_element_type=jnp.float32)
    a1_ref[...] = a1
    a2_ref[...] = a2
    bm1 = jnp.max(a1)
    bm2 = jnp.max(a2)

    @pl.when(i == 0)
    def _():
        msc[0] = bm1
        msc[1] = bm2

    @pl.when(i > 0)
    def _():
        msc[0] = jnp.maximum(msc[0], bm1)
        msc[1] = jnp.maximum(msc[1], bm2)

    m_ref[...] = jnp.maximum(msc[0] + msc[1], 0.0).reshape(1, 1)


def _tc_mid_body(p0_ref, p1_ref, s0_ref, s1_ref, bp_ref, w_ref, asr_ref,
                 adr_ref, h_ref, a1_ref, a2_ref, m_ref, msc):
    i = pl.program_id(0)
    s = s0_ref[...] + s1_ref[...] + 1e-16
    X = (p0_ref[...] + p1_ref[...]) / s + bp_ref[...]
    X = jnp.maximum(X, 0.0)
    h = jnp.dot(X, w_ref[...], preferred_element_type=jnp.float32)
    h_ref[...] = h
    a1 = jnp.dot(h, asr_ref[...], preferred_element_type=jnp.float32)
    a2 = jnp.dot(h, adr_ref[...], preferred_element_type=jnp.float32)
    a1_ref[...] = a1
    a2_ref[...] = a2
    bm1 = jnp.max(a1)
    bm2 = jnp.max(a2)

    @pl.when(i == 0)
    def _():
        msc[0] = bm1
        msc[1] = bm2

    @pl.when(i > 0)
    def _():
        msc[0] = jnp.maximum(msc[0], bm1)
        msc[1] = jnp.maximum(msc[1], bm2)

    m_ref[...] = jnp.maximum(msc[0] + msc[1], 0.0).reshape(1, 1)


def _tc_layer(X_or_parts, W, a_src, a_dst, first):
    d_in, d = W.shape
    BN = 1280
    grid = (N_EXT // BN,)
    out_shape = (
        jax.ShapeDtypeStruct((N_EXT, d), jnp.float32),
        jax.ShapeDtypeStruct((N_EXT, 1), jnp.float32),
        jax.ShapeDtypeStruct((N_EXT, 1), jnp.float32),
        jax.ShapeDtypeStruct((1, 1), jnp.float32),
    )
    out_specs = (
        pl.BlockSpec((BN, d), lambda i: (i, 0)),
        pl.BlockSpec((BN, 1), lambda i: (i, 0)),
        pl.BlockSpec((BN, 1), lambda i: (i, 0)),
        pl.BlockSpec((1, 1), lambda i: (0, 0)),
    )
    asr = a_src.reshape(d, 1)
    adr = a_dst.reshape(d, 1)
    if first:
        x = X_or_parts
        return pl.pallas_call(
            _tc_first_body,
            grid=grid,
            in_specs=[
                pl.BlockSpec((BN, d_in), lambda i: (i, 0)),
                pl.BlockSpec((d_in, d), lambda i: (0, 0)),
                pl.BlockSpec((d, 1), lambda i: (0, 0)),
                pl.BlockSpec((d, 1), lambda i: (0, 0)),
            ],
            out_specs=out_specs,
            out_shape=out_shape,
            scratch_shapes=[pltpu.SMEM((2,), jnp.float32)],
        )(x, W, asr, adr)
    p0, p1, s0, s1, bp = X_or_parts
    return pl.pallas_call(
        _tc_mid_body,
        grid=grid,
        in_specs=[
            pl.BlockSpec((BN, d_in), lambda i: (i, 0)),
            pl.BlockSpec((BN, d_in), lambda i: (i, 0)),
            pl.BlockSpec((BN, 1), lambda i: (i, 0)),
            pl.BlockSpec((BN, 1), lambda i: (i, 0)),
            pl.BlockSpec((1, d_in), lambda i: (0, 0)),
            pl.BlockSpec((d_in, d), lambda i: (0, 0)),
            pl.BlockSpec((d, 1), lambda i: (0, 0)),
            pl.BlockSpec((d, 1), lambda i: (0, 0)),
        ],
        out_specs=out_specs,
        out_shape=out_shape,
        scratch_shapes=[pltpu.SMEM((2,), jnp.float32)],
    )(p0, p1, s0.reshape(N_EXT, 1), s1.reshape(N_EXT, 1),
      bp.reshape(1, d_in), W, asr, adr)


def _tc_norm_body(p0_ref, p1_ref, s0_ref, s1_ref, b_ref, o_ref):
    s = s0_ref[...] + s1_ref[...] + 1e-16
    o_ref[...] = (p0_ref[...] + p1_ref[...]) / s + b_ref[...]


def _tc_norm(p0, p1, s0, s1, b):
    d = p0.shape[-1]
    BN = 1280
    return pl.pallas_call(
        _tc_norm_body,
        grid=(N_EXT // BN,),
        in_specs=[
            pl.BlockSpec((BN, d), lambda i: (i, 0)),
            pl.BlockSpec((BN, d), lambda i: (i, 0)),
            pl.BlockSpec((BN, 1), lambda i: (i, 0)),
            pl.BlockSpec((BN, 1), lambda i: (i, 0)),
            pl.BlockSpec((1, d), lambda i: (0, 0)),
        ],
        out_specs=pl.BlockSpec((BN, d), lambda i: (i, 0)),
        out_shape=jax.ShapeDtypeStruct((N_EXT, d), jnp.float32),
    )(p0, p1, s0.reshape(N_EXT, 1), s1.reshape(N_EXT, 1), b.reshape(1, d))


def _bcast_lane(vec, l):
    idx = jnp.full((16, 1), l, jnp.int32)
    return lax.gather(
        vec, idx,
        lax.GatherDimensionNumbers(
            offset_dims=(), collapsed_slice_dims=(0,), start_index_map=(0,)),
        slice_sizes=(1,),
        mode=lax.GatherScatterMode.PROMISE_IN_BOUNDS)

@functools.partial(jax.jit, static_argnames=("d",))
def _sc_edge(h, asv, adv, mvec, srcp, dstp, znd, zn, d):
    mesh = plsc.VectorSubcoreMesh(core_axis_name="c", subcore_axis_name="s")

    @functools.partial(
        pl.kernel,
        out_type=(
            jax.ShapeDtypeStruct((NC, N_EXT, d), jnp.float32),
            jax.ShapeDtypeStruct((NC, N_EXT), jnp.float32),
        ),
        mesh=mesh,
        scratch_types=[
            pltpu.VMEM((PK, B), jnp.int32),
            pltpu.VMEM((PK, B), jnp.int32),
            pltpu.VMEM((PK * B,), jnp.float32),
            pltpu.VMEM((PK * B,), jnp.float32),
            pltpu.VMEM((NCH * B,), jnp.float32),
            pltpu.VMEM((16,), jnp.float32),
            pltpu.VMEM((RD, B), jnp.int32),
            pltpu.VMEM((RD, B), jnp.int32),
            pltpu.VMEM((B, d), jnp.float32),
            pltpu.VMEM((B, d), jnp.float32),
            pltpu.VMEM_SHARED((N_EXT, d), jnp.float32),
            pltpu.VMEM_SHARED((N_EXT,), jnp.float32),
            pltpu.SemaphoreType.DMA,
            pltpu.SemaphoreType.DMA,
            pltpu.SemaphoreType.DMA,
            pltpu.SemaphoreType.DMA,
        ],
        compiler_params=pltpu.CompilerParams(use_tc_tiling_on_sc=False),
    )
    def k(h_hbm, as_hbm, ad_hbm, m_hbm, src_hbm, dst_hbm, znd_hbm, zn_hbm,
          acc_out, s_out, sidx1, didx1, ag_v, bg_v, ex_v, m_v,
          sidx2, didx2, rowsA, rowsB, acc_sh, s_sh,
          psem, isem, gA, gB):
        rows = (rowsA, rowsB)
        gsem = (gA, gB)
        cidx = lax.axis_index("c")
        sidx = lax.axis_index("s")
        wid = sidx * NC + cidx
        r0 = sidx * RPW

        pltpu.sync_copy(znd_hbm.at[pl.ds(r0, RPW)], acc_sh.at[pl.ds(r0, RPW)])

        @pl.when(sidx == 0)
        def _():
            pltpu.sync_copy(zn_hbm, s_sh)

        pltpu.sync_copy(m_hbm, m_v)
        plsc.subcore_barrier()

        mv = m_v[...]

        def p1(grp, carry):
            ch0 = grp * PK
            ci = pltpu.async_copy(src_hbm.at[wid, pl.ds(ch0, PK)], sidx1, isem)
            cj = pltpu.async_copy(dst_hbm.at[wid, pl.ds(ch0, PK)], didx1, isem)
            ci.wait()
            cj.wait()
            cps = []
            for kk in range(PK):
                cps.append(pltpu.async_copy(
                    as_hbm.at[sidx1.at[kk]], ag_v.at[pl.ds(kk * B, B)], psem))
                cps.append(pltpu.async_copy(
                    ad_hbm.at[didx1.at[kk]], bg_v.at[pl.ds(kk * B, B)], psem))
            for cp in cps:
                cp.wait()
            for kk in range(PK):
                for j in range(B // 16):
                    sl = pl.ds(kk * B + j * 16, 16)
                    e = ag_v[sl] + bg_v[sl]
                    e = jnp.where(e > 0.0, e, e * 0.2)
                    ex_v[pl.ds((ch0 + kk) * B + j * 16, 16)] = jnp.exp(e - mv)
            for kk in range(PK):
                pltpu.sync_copy(ex_v.at[pl.ds((ch0 + kk) * B, B)],
                                s_sh.at[didx1.at[kk]], add=True)
            return carry

        lax.fori_loop(0, NCH // PK, p1, 0)

        def _scale(buf, ch):
            for g in range(B // 16):
                exg = ex_v[pl.ds(ch * B + g * 16, 16)]
                for l in range(16):
                    j = g * 16 + l
                    exj = _bcast_lane(exg, l)
                    for f in range(d // 16):
                        slf = pl.ds(f * 16, 16)
                        buf[j, slf] = buf[j, slf] * exj

        def _idx_pair(chn, slot):
            pltpu.async_copy(
                src_hbm.at[wid, pl.ds(chn, 1)], sidx2.at[pl.ds(slot, 1)], isem)
            pltpu.async_copy(
                dst_hbm.at[wid, pl.ds(chn, 1)], didx2.at[pl.ds(slot, 1)], isem)

        def _idx_wait():
            pltpu.make_async_copy(
                src_hbm.at[wid, pl.ds(0, 1)], sidx2.at[pl.ds(0, 1)],
                isem).wait()
            pltpu.make_async_copy(
                dst_hbm.at[wid, pl.ds(0, 1)], didx2.at[pl.ds(0, 1)],
                isem).wait()

        _idx_pair(0, 0)
        _idx_wait()
        pltpu.async_copy(h_hbm.at[sidx2.at[0]], rows[0], gsem[0])
        _idx_pair(1, 1)

        def p2(qq, carry):
            for kq in range(4):
                ch = qq * 4 + kq
                nb = (kq + 1) % 2

                @pl.when(ch + 1 < NCH)
                def _():
                    _idx_wait()
                    pltpu.async_copy(
                        h_hbm.at[sidx2.at[(kq + 1) % RD]], rows[nb], gsem[nb])

                @pl.when(ch + 2 < NCH)
                def _():
                    _idx_pair(ch + 2, (kq + 2) % RD)

                pltpu.make_async_copy(
                    h_hbm.at[sidx2.at[0]], rows[kq % 2], gsem[kq % 2]).wait()
                _scale(rows[kq % 2], ch)
                pltpu.sync_copy(rows[kq % 2],
                                acc_sh.at[didx2.at[kq]], add=True)
            return carry

        lax.fori_loop(0, NCH // 4, p2, 0)

        plsc.subcore_barrier()

        pltpu.sync_copy(acc_sh.at[pl.ds(r0, RPW)],
                        acc_out.at[cidx, pl.ds(r0, RPW)])

        @pl.when(sidx == 0)
        def _():
            pltpu.sync_copy(s_sh, s_out.at[cidx])

    return k(h, asv, adv, mvec, srcp, dstp, znd, zn)


def kernel(x, edge_index, W1, a_src1, a_dst1, b1, W2, a_src2, a_dst2, b2,
           W3, a_src3, a_dst3, b3):
    src = edge_index[0].astype(jnp.int32)
    dst = edge_index[1].astype(jnp.int32)
    pad = E_PAD - E
    pad_idx = DUMMY + (jnp.arange(pad, dtype=jnp.int32) % (N_EXT - N))
    srcp = jnp.concatenate([src, pad_idx]).reshape(NW, NCH, B)
    dstp = jnp.concatenate([dst, pad_idx]).reshape(NW, NCH, B)
    x_ext = jnp.pad(x, ((0, N_EXT - N), (0, 0)))

    znd128 = jnp.zeros((N_EXT, 128), jnp.float32)
    znd64 = jnp.zeros((N_EXT, 64), jnp.float32)
    zn = jnp.zeros((N_EXT,), jnp.float32)

    h, a1, a2, m = _tc_layer(x_ext, W1, a_src1, a_dst1, first=True)
    mv = jnp.full((16,), m[0, 0], jnp.float32)
    acc, s = _sc_edge(h, a1.reshape(N_EXT), a2.reshape(N_EXT), mv,
                      srcp, dstp, znd128, zn, d=128)

    h, a1, a2, m = _tc_layer(
        (acc[0], acc[1], s[0], s[1], b1), W2, a_src2, a_dst2, first=False)
    mv = jnp.full((16,), m[0, 0], jnp.float32)
    acc, s = _sc_edge(h, a1.reshape(N_EXT), a2.reshape(N_EXT), mv,
                      srcp, dstp, znd64, zn, d=64)

    h, a1, a2, m = _tc_layer(
        (acc[0], acc[1], s[0], s[1], b2), W3, a_src3, a_dst3, first=False)
    mv = jnp.full((16,), m[0, 0], jnp.float32)
    acc, s = _sc_edge(h, a1.reshape(N_EXT), a2.reshape(N_EXT), mv,
                      srcp, dstp, znd64, zn, d=64)

    out = _tc_norm(acc[0], acc[1], s[0], s[1], b3)
    return out[:N]


# speedup vs baseline: 2.3274x; 1.2823x over previous
import functools

import jax
import jax.numpy as jnp
from jax import lax
from jax.experimental import pallas as pl
from jax.experimental.pallas import tpu as pltpu
from jax.experimental.pallas import tpu_sc as plsc

N = 10000
N_EXT = 10240
E = 320000
NC, NS = 2, 16
NW = NC * NS
B = 128
NCH = 80
EPW = NCH * B
E_PAD = NW * EPW
RPW = N_EXT // NS
DUMMY = N
PK = 4
RD = 4


def _tc_first_body(x_ref, w_ref, asr_ref, adr_ref,
                   h_ref, a1_ref, a2_ref, m_ref, msc):
    i = pl.program_id(0)
    h = jnp.dot(x_ref[...], w_ref[...], preferred_element_type=jnp.float32)
    h_ref[...] = h
    a1 = jnp.dot(h, asr_ref[...], preferred_element_type=jnp.float32)
    a2 = jnp.dot(h, adr_ref[...], preferred_element_type=jnp.float32)
    a1_ref[...] = a1
    a2_ref[...] = a2
    bm1 = jnp.max(a1)
    bm2 = jnp.max(a2)

    @pl.when(i == 0)
    def _():
        msc[0] = bm1
        msc[1] = bm2

    @pl.when(i > 0)
    def _():
        msc[0] = jnp.maximum(msc[0], bm1)
        msc[1] = jnp.maximum(msc[1], bm2)

    m_ref[...] = jnp.maximum(msc[0] + msc[1], 0.0).reshape(1, 1)


def _tc_mid_body(p0_ref, p1_ref, s0_ref, s1_ref, bp_ref, w_ref, asr_ref,
                 adr_ref, h_ref, a1_ref, a2_ref, m_ref, msc):
    i = pl.program_id(0)
    s = s0_ref[...] + s1_ref[...] + 1e-16
    X = (p0_ref[...] + p1_ref[...]) / s + bp_ref[...]
    X = jnp.maximum(X, 0.0)
    h = jnp.dot(X, w_ref[...], preferred_element_type=jnp.float32)
    h_ref[...] = h
    a1 = jnp.dot(h, asr_ref[...], preferred_element_type=jnp.float32)
    a2 = jnp.dot(h, adr_ref[...], preferred_element_type=jnp.float32)
    a1_ref[...] = a1
    a2_ref[...] = a2
    bm1 = jnp.max(a1)
    bm2 = jnp.max(a2)

    @pl.when(i == 0)
    def _():
        msc[0] = bm1
        msc[1] = bm2

    @pl.when(i > 0)
    def _():
        msc[0] = jnp.maximum(msc[0], bm1)
        msc[1] = jnp.maximum(msc[1], bm2)

    m_ref[...] = jnp.maximum(msc[0] + msc[1], 0.0).reshape(1, 1)


def _tc_layer(X_or_parts, W, a_src, a_dst, first):
    d_in, d = W.shape
    BN = 1280
    grid = (N_EXT // BN,)
    out_shape = (
        jax.ShapeDtypeStruct((N_EXT, d), jnp.float32),
        jax.ShapeDtypeStruct((N_EXT, 1), jnp.float32),
        jax.ShapeDtypeStruct((N_EXT, 1), jnp.float32),
        jax.ShapeDtypeStruct((1, 1), jnp.float32),
    )
    out_specs = (
        pl.BlockSpec((BN, d), lambda i: (i, 0)),
        pl.BlockSpec((BN, 1), lambda i: (i, 0)),
        pl.BlockSpec((BN, 1), lambda i: (i, 0)),
        pl.BlockSpec((1, 1), lambda i: (0, 0)),
    )
    asr = a_src.reshape(d, 1)
    adr = a_dst.reshape(d, 1)
    if first:
        x = X_or_parts
        return pl.pallas_call(
            _tc_first_body,
            grid=grid,
            in_specs=[
                pl.BlockSpec((BN, d_in), lambda i: (i, 0)),
                pl.BlockSpec((d_in, d), lambda i: (0, 0)),
                pl.BlockSpec((d, 1), lambda i: (0, 0)),
                pl.BlockSpec((d, 1), lambda i: (0, 0)),
            ],
            out_specs=out_specs,
            out_shape=out_shape,
            scratch_shapes=[pltpu.SMEM((2,), jnp.float32)],
        )(x, W, asr, adr)
    p0, p1, s0, s1, bp = X_or_parts
    return pl.pallas_call(
        _tc_mid_body,
        grid=grid,
        in_specs=[
            pl.BlockSpec((BN, d_in), lambda i: (i, 0)),
            pl.BlockSpec((BN, d_in), lambda i: (i, 0)),
            pl.BlockSpec((BN, 1), lambda i: (i, 0)),
            pl.BlockSpec((BN, 1), lambda i: (i, 0)),
            pl.BlockSpec((1, d_in), lambda i: (0, 0)),
            pl.BlockSpec((d_in, d), lambda i: (0, 0)),
            pl.BlockSpec((d, 1), lambda i: (0, 0)),
            pl.BlockSpec((d, 1), lambda i: (0, 0)),
        ],
        out_specs=out_specs,
        out_shape=out_shape,
        scratch_shapes=[pltpu.SMEM((2,), jnp.float32)],
    )(p0, p1, s0.reshape(N_EXT, 1), s1.reshape(N_EXT, 1),
      bp.reshape(1, d_in), W, asr, adr)


def _tc_norm_body(p0_ref, p1_ref, s0_ref, s1_ref, b_ref, o_ref):
    s = s0_ref[...] + s1_ref[...] + 1e-16
    o_ref[...] = (p0_ref[...] + p1_ref[...]) / s + b_ref[...]


def _tc_norm(p0, p1, s0, s1, b):
    d = p0.shape[-1]
    BN = 1280
    return pl.pallas_call(
        _tc_norm_body,
        grid=(N_EXT // BN,),
        in_specs=[
            pl.BlockSpec((BN, d), lambda i: (i, 0)),
            pl.BlockSpec((BN, d), lambda i: (i, 0)),
            pl.BlockSpec((BN, 1), lambda i: (i, 0)),
            pl.BlockSpec((BN, 1), lambda i: (i, 0)),
            pl.BlockSpec((1, d), lambda i: (0, 0)),
        ],
        out_specs=pl.BlockSpec((BN, d), lambda i: (i, 0)),
        out_shape=jax.ShapeDtypeStruct((N_EXT, d), jnp.float32),
    )(p0, p1, s0.reshape(N_EXT, 1), s1.reshape(N_EXT, 1), b.reshape(1, d))


def _bcast_lane(vec, l):
    idx = jnp.full((16, 1), l, jnp.int32)
    return lax.gather(
        vec, idx,
        lax.GatherDimensionNumbers(
            offset_dims=(), collapsed_slice_dims=(0,), start_index_map=(0,)),
        slice_sizes=(1,),
        mode=lax.GatherScatterMode.PROMISE_IN_BOUNDS)

@functools.partial(jax.jit, static_argnames=("d",))
def _sc_edge(h, asv, adv, mvec, srcp, dstp, znd, zn, d):
    mesh = plsc.VectorSubcoreMesh(core_axis_name="c", subcore_axis_name="s")

    @functools.partial(
        pl.kernel,
        out_type=(
            jax.ShapeDtypeStruct((NC, N_EXT, d), jnp.float32),
            jax.ShapeDtypeStruct((NC, N_EXT), jnp.float32),
        ),
        mesh=mesh,
        scratch_types=[
            pltpu.VMEM((RD, B), jnp.int32),
            pltpu.VMEM((RD, B), jnp.int32),
            pltpu.VMEM((B,), jnp.float32),
            pltpu.VMEM((B,), jnp.float32),
            pltpu.VMEM((B,), jnp.float32),
            pltpu.VMEM((B,), jnp.float32),
            pltpu.VMEM((B,), jnp.float32),
            pltpu.VMEM((B,), jnp.float32),
            pltpu.VMEM((16,), jnp.float32),
            pltpu.VMEM((B, d), jnp.float32),
            pltpu.VMEM((B, d), jnp.float32),
            pltpu.VMEM_SHARED((N_EXT, d), jnp.float32),
            pltpu.VMEM_SHARED((N_EXT,), jnp.float32),
            pltpu.SemaphoreType.DMA,
            pltpu.SemaphoreType.DMA,
            pltpu.SemaphoreType.DMA,
            pltpu.SemaphoreType.DMA,
            pltpu.SemaphoreType.DMA,
        ],
        compiler_params=pltpu.CompilerParams(use_tc_tiling_on_sc=False),
    )
    def k(h_hbm, as_hbm, ad_hbm, m_hbm, src_hbm, dst_hbm, znd_hbm, zn_hbm,
          acc_out, s_out, sidx2, didx2, agA, agB, bgA, bgB, exA, exB, m_v,
          rowsA, rowsB, acc_sh, s_sh, isem, aA, aB, gA, gB):
        rows = (rowsA, rowsB)
        ag = (agA, agB)
        bg = (bgA, bgB)
        exv = (exA, exB)
        asem = (aA, aB)
        gsem = (gA, gB)
        cidx = lax.axis_index("c")
        sidx = lax.axis_index("s")
        wid = sidx * NC + cidx
        r0 = sidx * RPW

        pltpu.sync_copy(znd_hbm.at[pl.ds(r0, RPW)], acc_sh.at[pl.ds(r0, RPW)])

        @pl.when(sidx == 0)
        def _():
            pltpu.sync_copy(zn_hbm, s_sh)

        pltpu.sync_copy(m_hbm, m_v)
        plsc.subcore_barrier()

        mv = m_v[...]

        def _idx_pair(chn, slot):
            pltpu.async_copy(
                src_hbm.at[wid, pl.ds(chn, 1)], sidx2.at[pl.ds(slot, 1)], isem)
            pltpu.async_copy(
                dst_hbm.at[wid, pl.ds(chn, 1)], didx2.at[pl.ds(slot, 1)], isem)

        def _idx_wait():
            pltpu.make_async_copy(
                src_hbm.at[wid, pl.ds(0, 1)], sidx2.at[pl.ds(0, 1)],
                isem).wait()
            pltpu.make_async_copy(
                dst_hbm.at[wid, pl.ds(0, 1)], didx2.at[pl.ds(0, 1)],
                isem).wait()

        def _fire(slot, par):
            pltpu.async_copy(as_hbm.at[sidx2.at[slot]], ag[par], asem[par])
            pltpu.async_copy(ad_hbm.at[didx2.at[slot]], bg[par], asem[par])
            pltpu.async_copy(h_hbm.at[sidx2.at[slot]], rows[par], gsem[par])

        def _wait_scalars(par):
            pltpu.make_async_copy(as_hbm.at[sidx2.at[0]], ag[par],
                                  asem[par]).wait()
            pltpu.make_async_copy(ad_hbm.at[didx2.at[0]], bg[par],
                                  asem[par]).wait()

        def _wait_rows(par):
            pltpu.make_async_copy(h_hbm.at[sidx2.at[0]], rows[par],
                                  gsem[par]).wait()

        def _scale(buf, exb):
            for g in range(B // 16):
                exg = exb[pl.ds(g * 16, 16)]
                for l in range(16):
                    j = g * 16 + l
                    exj = _bcast_lane(exg, l)
                    for f in range(d // 16):
                        slf = pl.ds(f * 16, 16)
                        buf[j, slf] = buf[j, slf] * exj

        _idx_pair(0, 0)
        _idx_wait()
        _fire(0, 0)
        _idx_pair(1, 1)

        def p2(qq, carry):
            for kq in range(4):
                ch = qq * 4 + kq
                par = kq % 2

                @pl.when(ch + 1 < NCH)
                def _():
                    _idx_wait()
                    _fire((kq + 1) % RD, 1 - par)

                @pl.when(ch + 2 < NCH)
                def _():
                    _idx_pair(ch + 2, (kq + 2) % RD)

                _wait_scalars(par)
                for g in range(B // 16):
                    sl = pl.ds(g * 16, 16)
                    e = ag[par][sl] + bg[par][sl]
                    e = jnp.where(e > 0.0, e, e * 0.2)
                    exv[par][sl] = jnp.exp(e - mv)
                pltpu.sync_copy(exv[par], s_sh.at[didx2.at[kq]], add=True)
                _wait_rows(par)
                _scale(rows[par], exv[par])
                pltpu.sync_copy(rows[par],
                                acc_sh.at[didx2.at[kq]], add=True)
            return carry

        lax.fori_loop(0, NCH // 4, p2, 0)

        plsc.subcore_barrier()

        pltpu.sync_copy(acc_sh.at[pl.ds(r0, RPW)],
                        acc_out.at[cidx, pl.ds(r0, RPW)])

        @pl.when(sidx == 0)
        def _():
            pltpu.sync_copy(s_sh, s_out.at[cidx])

    return k(h, asv, adv, mvec, srcp, dstp, znd, zn)


def kernel(x, edge_index, W1, a_src1, a_dst1, b1, W2, a_src2, a_dst2, b2,
           W3, a_src3, a_dst3, b3):
    src = edge_index[0].astype(jnp.int32)
    dst = edge_index[1].astype(jnp.int32)
    pad = E_PAD - E
    pad_idx = DUMMY + (jnp.arange(pad, dtype=jnp.int32) % (N_EXT - N))
    srcp = jnp.concatenate([src, pad_idx]).reshape(NW, NCH, B)
    dstp = jnp.concatenate([dst, pad_idx]).reshape(NW, NCH, B)
    x_ext = jnp.pad(x, ((0, N_EXT - N), (0, 0)))

    znd128 = jnp.zeros((N_EXT, 128), jnp.float32)
    znd64 = jnp.zeros((N_EXT, 64), jnp.float32)
    zn = jnp.zeros((N_EXT,), jnp.float32)

    h, a1, a2, m = _tc_layer(x_ext, W1, a_src1, a_dst1, first=True)
    mv = jnp.full((16,), m[0, 0], jnp.float32)
    acc, s = _sc_edge(h, a1.reshape(N_EXT), a2.reshape(N_EXT), mv,
                      srcp, dstp, znd128, zn, d=128)

    h, a1, a2, m = _tc_layer(
        (acc[0], acc[1], s[0], s[1], b1), W2, a_src2, a_dst2, first=False)
    mv = jnp.full((16,), m[0, 0], jnp.float32)
    acc, s = _sc_edge(h, a1.reshape(N_EXT), a2.reshape(N_EXT), mv,
                      srcp, dstp, znd64, zn, d=64)

    h, a1, a2, m = _tc_layer(
        (acc[0], acc[1], s[0], s[1], b2), W3, a_src3, a_dst3, first=False)
    mv = jnp.full((16,), m[0, 0], jnp.float32)
    acc, s = _sc_edge(h, a1.reshape(N_EXT), a2.reshape(N_EXT), mv,
                      srcp, dstp, znd64, zn, d=64)

    out = _tc_norm(acc[0], acc[1], s[0], s[1], b3)
    return out[:N]


# speedup vs baseline: 2.3303x; 1.0012x over previous
import functools

import jax
import jax.numpy as jnp
from jax import lax
from jax.experimental import pallas as pl
from jax.experimental.pallas import tpu as pltpu
from jax.experimental.pallas import tpu_sc as plsc

N = 10000
N_EXT = 10240
E = 320000
NC, NS = 2, 16
NW = NC * NS
B = 128
NCH = 80
EPW = NCH * B
E_PAD = NW * EPW
RPW = N_EXT // NS
DUMMY = N
PK = 4
RD = 4


def _tc_first_body(x_ref, w_ref, asr_ref, adr_ref,
                   h_ref, a1_ref, a2_ref, m_ref, msc):
    i = pl.program_id(0)
    h = jnp.dot(x_ref[...], w_ref[...], preferred_element_type=jnp.float32)
    h_ref[...] = h
    a1 = jnp.dot(h, asr_ref[...], preferred_element_type=jnp.float32)
    a2 = jnp.dot(h, adr_ref[...], preferred_element_type=jnp.float32)
    a1_ref[...] = a1
    a2_ref[...] = a2
    bm1 = jnp.max(a1)
    bm2 = jnp.max(a2)

    @pl.when(i == 0)
    def _():
        msc[0] = bm1
        msc[1] = bm2

    @pl.when(i > 0)
    def _():
        msc[0] = jnp.maximum(msc[0], bm1)
        msc[1] = jnp.maximum(msc[1], bm2)

    m_ref[...] = jnp.full((1, 16), jnp.maximum(msc[0] + msc[1], 0.0))


def _tc_mid_body(pp_ref, ss_ref, bp_ref, w_ref, asr_ref,
                 adr_ref, h_ref, a1_ref, a2_ref, m_ref, msc):
    i = pl.program_id(0)
    s = ss_ref[0] + ss_ref[1] + 1e-16
    X = (pp_ref[0] + pp_ref[1]) / s + bp_ref[...]
    X = jnp.maximum(X, 0.0)
    h = jnp.dot(X, w_ref[...], preferred_element_type=jnp.float32)
    h_ref[...] = h
    a1 = jnp.dot(h, asr_ref[...], preferred_element_type=jnp.float32)
    a2 = jnp.dot(h, adr_ref[...], preferred_element_type=jnp.float32)
    a1_ref[...] = a1
    a2_ref[...] = a2
    bm1 = jnp.max(a1)
    bm2 = jnp.max(a2)

    @pl.when(i == 0)
    def _():
        msc[0] = bm1
        msc[1] = bm2

    @pl.when(i > 0)
    def _():
        msc[0] = jnp.maximum(msc[0], bm1)
        msc[1] = jnp.maximum(msc[1], bm2)

    m_ref[...] = jnp.full((1, 16), jnp.maximum(msc[0] + msc[1], 0.0))


def _tc_layer(X_or_parts, W, a_src, a_dst, first):
    d_in, d = W.shape
    BN = 1280
    grid = (N_EXT // BN,)
    out_shape = (
        jax.ShapeDtypeStruct((N_EXT, d), jnp.float32),
        jax.ShapeDtypeStruct((N_EXT, 1), jnp.float32),
        jax.ShapeDtypeStruct((N_EXT, 1), jnp.float32),
        jax.ShapeDtypeStruct((1, 16), jnp.float32),
    )
    out_specs = (
        pl.BlockSpec((BN, d), lambda i: (i, 0)),
        pl.BlockSpec((BN, 1), lambda i: (i, 0)),
        pl.BlockSpec((BN, 1), lambda i: (i, 0)),
        pl.BlockSpec((1, 16), lambda i: (0, 0)),
    )
    asr = a_src.reshape(d, 1)
    adr = a_dst.reshape(d, 1)
    if first:
        x = X_or_parts
        return pl.pallas_call(
            _tc_first_body,
            grid=grid,
            in_specs=[
                pl.BlockSpec((BN, d_in), lambda i: (i, 0)),
                pl.BlockSpec((d_in, d), lambda i: (0, 0)),
                pl.BlockSpec((d, 1), lambda i: (0, 0)),
                pl.BlockSpec((d, 1), lambda i: (0, 0)),
            ],
            out_specs=out_specs,
            out_shape=out_shape,
            scratch_shapes=[pltpu.SMEM((2,), jnp.float32)],
        )(x, W, asr, adr)
    pp, ss, bp = X_or_parts
    return pl.pallas_call(
        _tc_mid_body,
        grid=grid,
        in_specs=[
            pl.BlockSpec((2, BN, d_in), lambda i: (0, i, 0)),
            pl.BlockSpec((2, BN, 1), lambda i: (0, i, 0)),
            pl.BlockSpec((1, d_in), lambda i: (0, 0)),
            pl.BlockSpec((d_in, d), lambda i: (0, 0)),
            pl.BlockSpec((d, 1), lambda i: (0, 0)),
            pl.BlockSpec((d, 1), lambda i: (0, 0)),
        ],
        out_specs=out_specs,
        out_shape=out_shape,
        scratch_shapes=[pltpu.SMEM((2,), jnp.float32)],
    )(pp, ss.reshape(NC, N_EXT, 1), bp.reshape(1, d_in), W, asr, adr)


def _tc_norm_body(pp_ref, ss_ref, b_ref, o_ref):
    s = ss_ref[0] + ss_ref[1] + 1e-16
    o_ref[...] = (pp_ref[0] + pp_ref[1]) / s + b_ref[...]


def _tc_norm(pp, ss, b):
    d = pp.shape[-1]
    BN = 1280
    return pl.pallas_call(
        _tc_norm_body,
        grid=(N_EXT // BN,),
        in_specs=[
            pl.BlockSpec((2, BN, d), lambda i: (0, i, 0)),
            pl.BlockSpec((2, BN, 1), lambda i: (0, i, 0)),
            pl.BlockSpec((1, d), lambda i: (0, 0)),
        ],
        out_specs=pl.BlockSpec((BN, d), lambda i: (i, 0)),
        out_shape=jax.ShapeDtypeStruct((N_EXT, d), jnp.float32),
    )(pp, ss.reshape(NC, N_EXT, 1), b.reshape(1, d))


def _bcast_lane(vec, l):
    idx = jnp.full((16, 1), l, jnp.int32)
    return lax.gather(
        vec, idx,
        lax.GatherDimensionNumbers(
            offset_dims=(), collapsed_slice_dims=(0,), start_index_map=(0,)),
        slice_sizes=(1,),
        mode=lax.GatherScatterMode.PROMISE_IN_BOUNDS)

@functools.partial(jax.jit, static_argnames=("d",))
def _sc_edge(h, asv, adv, mvec, srcp, dstp, znd, zn, d):
    mesh = plsc.VectorSubcoreMesh(core_axis_name="c", subcore_axis_name="s")

    @functools.partial(
        pl.kernel,
        out_type=(
            jax.ShapeDtypeStruct((NC, N_EXT, d), jnp.float32),
            jax.ShapeDtypeStruct((NC, N_EXT), jnp.float32),
        ),
        mesh=mesh,
        scratch_types=[
            pltpu.VMEM((RD, B), jnp.int32),
            pltpu.VMEM((RD, B), jnp.int32),
            pltpu.VMEM((B,), jnp.float32),
            pltpu.VMEM((B,), jnp.float32),
            pltpu.VMEM((B,), jnp.float32),
            pltpu.VMEM((B,), jnp.float32),
            pltpu.VMEM((B,), jnp.float32),
            pltpu.VMEM((B,), jnp.float32),
            pltpu.VMEM((16,), jnp.float32),
            pltpu.VMEM((B, d), jnp.float32),
            pltpu.VMEM((B, d), jnp.float32),
            pltpu.VMEM_SHARED((N_EXT, d), jnp.float32),
            pltpu.VMEM_SHARED((N_EXT,), jnp.float32),
            (pltpu.VMEM_SHARED((N_EXT, d), jnp.float32)
             if d <= 64 else pltpu.VMEM_SHARED((8, d), jnp.float32)),
            pltpu.SemaphoreType.DMA,
            pltpu.SemaphoreType.DMA,
            pltpu.SemaphoreType.DMA,
            pltpu.SemaphoreType.DMA,
            pltpu.SemaphoreType.DMA,
            pltpu.SemaphoreType.DMA,
            pltpu.SemaphoreType.DMA,
            pltpu.SemaphoreType.DMA,
            pltpu.SemaphoreType.DMA,
        ],
        compiler_params=pltpu.CompilerParams(use_tc_tiling_on_sc=False),
    )
    def k(h_hbm, as_hbm, ad_hbm, m_hbm, src_hbm, dst_hbm, znd_hbm, zn_hbm,
          acc_out, s_out, sidx2, didx2, agA, agB, bgA, bgB, exA, exB, m_v,
          rowsA, rowsB, acc_sh, s_sh, h_spm, isem, aA, aB, gA, gB,
          zA, zB, wA, wB):
        rows = (rowsA, rowsB)
        ag = (agA, agB)
        bg = (bgA, bgB)
        exv = (exA, exB)
        asem = (aA, aB)
        gsem = (gA, gB)
        zsem = (zA, zB)
        wsem = (wA, wB)
        cidx = lax.axis_index("c")
        sidx = lax.axis_index("s")
        wid = sidx * NC + cidx
        r0 = sidx * RPW

        pltpu.sync_copy(znd_hbm.at[pl.ds(r0, RPW)], acc_sh.at[pl.ds(r0, RPW)])
        if d <= 64:
            pltpu.sync_copy(h_hbm.at[pl.ds(r0, RPW)],
                            h_spm.at[pl.ds(r0, RPW)])

        @pl.when(sidx == 0)
        def _():
            pltpu.sync_copy(zn_hbm, s_sh)

        pltpu.sync_copy(m_hbm, m_v)
        plsc.subcore_barrier()

        mv = m_v[...]

        def _idx_pair(chn, slot):
            pltpu.async_copy(
                src_hbm.at[wid, pl.ds(chn, 1)], sidx2.at[pl.ds(slot, 1)], isem)
            pltpu.async_copy(
                dst_hbm.at[wid, pl.ds(chn, 1)], didx2.at[pl.ds(slot, 1)], isem)

        def _idx_wait():
            pltpu.make_async_copy(
                src_hbm.at[wid, pl.ds(0, 1)], sidx2.at[pl.ds(0, 1)],
                isem).wait()
            pltpu.make_async_copy(
                dst_hbm.at[wid, pl.ds(0, 1)], didx2.at[pl.ds(0, 1)],
                isem).wait()

        h_tab = h_spm if d <= 64 else h_hbm

        def _fire(slot, par):
            pltpu.async_copy(as_hbm.at[sidx2.at[slot]], ag[par], asem[par])
            pltpu.async_copy(ad_hbm.at[didx2.at[slot]], bg[par], asem[par])
            pltpu.async_copy(h_tab.at[sidx2.at[slot]], rows[par], gsem[par])

        def _wait_scalars(par):
            pltpu.make_async_copy(as_hbm.at[sidx2.at[0]], ag[par],
                                  asem[par]).wait()
            pltpu.make_async_copy(ad_hbm.at[didx2.at[0]], bg[par],
                                  asem[par]).wait()

        def _wait_rows(par):
            pltpu.make_async_copy(h_tab.at[sidx2.at[0]], rows[par],
                                  gsem[par]).wait()

        def _drain_scatters(par):
            pltpu.make_async_copy(exv[par], s_sh.at[didx2.at[0]],
                                  zsem[par]).wait()
            pltpu.make_async_copy(rows[par], acc_sh.at[didx2.at[0]],
                                  wsem[par]).wait()

        def _scale(buf, exb):
            for g in range(B // 16):
                exg = exb[pl.ds(g * 16, 16)]
                for l in range(16):
                    j = g * 16 + l
                    exj = _bcast_lane(exg, l)
                    for f in range(d // 16):
                        slf = pl.ds(f * 16, 16)
                        buf[j, slf] = buf[j, slf] * exj

        _idx_pair(0, 0)
        _idx_wait()
        _fire(0, 0)
        _idx_pair(1, 1)

        def p2(qq, carry):
            for kq in range(4):
                ch = qq * 4 + kq
                par = kq % 2

                _wait_scalars(par)
                for g in range(B // 16):
                    sl = pl.ds(g * 16, 16)
                    e = ag[par][sl] + bg[par][sl]
                    e = jnp.where(e > 0.0, e, e * 0.2)
                    exv[par][sl] = jnp.exp(e - mv)

                @pl.when(ch + 1 < NCH)
                def _():
                    _idx_wait()

                    @pl.when(ch >= 1)
                    def _():
                        _drain_scatters(1 - par)

                    _fire((kq + 1) % RD, 1 - par)

                @pl.when(ch + 2 < NCH)
                def _():
                    _idx_pair(ch + 2, (kq + 2) % RD)

                pltpu.async_copy(exv[par], s_sh.at[didx2.at[kq]],
                                 zsem[par], add=True)
                _wait_rows(par)
                _scale(rows[par], exv[par])
                pltpu.async_copy(rows[par], acc_sh.at[didx2.at[kq]],
                                 wsem[par], add=True)
            return carry

        lax.fori_loop(0, NCH // 4, p2, 0)

        _drain_scatters(0)
        _drain_scatters(1)
        plsc.subcore_barrier()

        pltpu.sync_copy(acc_sh.at[pl.ds(r0, RPW)],
                        acc_out.at[cidx, pl.ds(r0, RPW)])

        @pl.when(sidx == 0)
        def _():
            pltpu.sync_copy(s_sh, s_out.at[cidx])

    return k(h, asv, adv, mvec, srcp, dstp, znd, zn)


def kernel(x, edge_index, W1, a_src1, a_dst1, b1, W2, a_src2, a_dst2, b2,
           W3, a_src3, a_dst3, b3):
    src = edge_index[0].astype(jnp.int32)
    dst = edge_index[1].astype(jnp.int32)
    pad = E_PAD - E
    pad_idx = DUMMY + (jnp.arange(pad, dtype=jnp.int32) % (N_EXT - N))
    srcp = jnp.concatenate([src, pad_idx]).reshape(NW, NCH, B)
    dstp = jnp.concatenate([dst, pad_idx]).reshape(NW, NCH, B)
    x_ext = jnp.pad(x, ((0, N_EXT - N), (0, 0)))

    znd128 = jnp.zeros((N_EXT, 128), jnp.float32)
    znd64 = jnp.zeros((N_EXT, 64), jnp.float32)
    zn = jnp.zeros((N_EXT,), jnp.float32)

    h, a1, a2, m = _tc_layer(x_ext, W1, a_src1, a_dst1, first=True)
    acc, s = _sc_edge(h, a1.reshape(N_EXT), a2.reshape(N_EXT),
                      m.reshape(16), srcp, dstp, znd128, zn, d=128)

    h, a1, a2, m = _tc_layer((acc, s, b1), W2, a_src2, a_dst2, first=False)
    acc, s = _sc_edge(h, a1.reshape(N_EXT), a2.reshape(N_EXT),
                      m.reshape(16), srcp, dstp, znd64, zn, d=64)

    h, a1, a2, m = _tc_layer((acc, s, b2), W3, a_src3, a_dst3, first=False)
    acc, s = _sc_edge(h, a1.reshape(N_EXT), a2.reshape(N_EXT),
                      m.reshape(16), srcp, dstp, znd64, zn, d=64)

    out = _tc_norm(acc, s, b3)
    return out[:N]


# speedup vs baseline: 2.6064x; 1.1185x over previous
import functools

import jax
import jax.numpy as jnp
from jax import lax
from jax.experimental import pallas as pl
from jax.experimental.pallas import tpu as pltpu
from jax.experimental.pallas import tpu_sc as plsc

N = 10000
N_EXT = 10240
E = 320000
NC, NS = 2, 16
NW = NC * NS
B = 128
NCH = 80
EPW = NCH * B
E_PAD = NW * EPW
RPW = N_EXT // NS
DUMMY = N
PK = 4
RD = 4


def _tc_first_body(x_ref, w_ref, asr_ref, adr_ref,
                   h_ref, a1_ref, a2_ref, m_ref, msc):
    i = pl.program_id(0)
    h = jnp.dot(x_ref[...], w_ref[...], preferred_element_type=jnp.float32)
    h_ref[...] = h
    a1 = jnp.dot(h, asr_ref[...], preferred_element_type=jnp.float32)
    a2 = jnp.dot(h, adr_ref[...], preferred_element_type=jnp.float32)
    a1_ref[...] = a1
    a2_ref[...] = a2
    bm1 = jnp.max(a1)
    bm2 = jnp.max(a2)

    @pl.when(i == 0)
    def _():
        msc[0] = bm1
        msc[1] = bm2

    @pl.when(i > 0)
    def _():
        msc[0] = jnp.maximum(msc[0], bm1)
        msc[1] = jnp.maximum(msc[1], bm2)

    m_ref[...] = jnp.full((1, 16), jnp.maximum(msc[0] + msc[1], 0.0))


def _tc_mid_body(pp_ref, ss_ref, bp_ref, w_ref, asr_ref,
                 adr_ref, h_ref, a1_ref, a2_ref, m_ref, msc):
    i = pl.program_id(0)
    s = ss_ref[0] + ss_ref[1] + 1e-16
    X = (pp_ref[0] + pp_ref[1]) / s + bp_ref[...]
    X = jnp.maximum(X, 0.0)
    h = jnp.dot(X, w_ref[...], preferred_element_type=jnp.float32)
    h_ref[...] = h
    a1 = jnp.dot(h, asr_ref[...], preferred_element_type=jnp.float32)
    a2 = jnp.dot(h, adr_ref[...], preferred_element_type=jnp.float32)
    a1_ref[...] = a1
    a2_ref[...] = a2
    bm1 = jnp.max(a1)
    bm2 = jnp.max(a2)

    @pl.when(i == 0)
    def _():
        msc[0] = bm1
        msc[1] = bm2

    @pl.when(i > 0)
    def _():
        msc[0] = jnp.maximum(msc[0], bm1)
        msc[1] = jnp.maximum(msc[1], bm2)

    m_ref[...] = jnp.full((1, 16), jnp.maximum(msc[0] + msc[1], 0.0))


def _tc_layer(X_or_parts, W, a_src, a_dst, first):
    d_in, d = W.shape
    BN = 1280
    grid = (N_EXT // BN,)
    out_shape = (
        jax.ShapeDtypeStruct((N_EXT, d), jnp.float32),
        jax.ShapeDtypeStruct((N_EXT, 1), jnp.float32),
        jax.ShapeDtypeStruct((N_EXT, 1), jnp.float32),
        jax.ShapeDtypeStruct((1, 16), jnp.float32),
    )
    out_specs = (
        pl.BlockSpec((BN, d), lambda i: (i, 0)),
        pl.BlockSpec((BN, 1), lambda i: (i, 0)),
        pl.BlockSpec((BN, 1), lambda i: (i, 0)),
        pl.BlockSpec((1, 16), lambda i: (0, 0)),
    )
    asr = a_src.reshape(d, 1)
    adr = a_dst.reshape(d, 1)
    if first:
        x = X_or_parts
        return pl.pallas_call(
            _tc_first_body,
            grid=grid,
            in_specs=[
                pl.BlockSpec((BN, d_in), lambda i: (i, 0)),
                pl.BlockSpec((d_in, d), lambda i: (0, 0)),
                pl.BlockSpec((d, 1), lambda i: (0, 0)),
                pl.BlockSpec((d, 1), lambda i: (0, 0)),
            ],
            out_specs=out_specs,
            out_shape=out_shape,
            scratch_shapes=[pltpu.SMEM((2,), jnp.float32)],
        )(x, W, asr, adr)
    pp, ss, bp = X_or_parts
    return pl.pallas_call(
        _tc_mid_body,
        grid=grid,
        in_specs=[
            pl.BlockSpec((2, BN, d_in), lambda i: (0, i, 0)),
            pl.BlockSpec((2, BN, 1), lambda i: (0, i, 0)),
            pl.BlockSpec((1, d_in), lambda i: (0, 0)),
            pl.BlockSpec((d_in, d), lambda i: (0, 0)),
            pl.BlockSpec((d, 1), lambda i: (0, 0)),
            pl.BlockSpec((d, 1), lambda i: (0, 0)),
        ],
        out_specs=out_specs,
        out_shape=out_shape,
        scratch_shapes=[pltpu.SMEM((2,), jnp.float32)],
    )(pp, ss.reshape(NC, N_EXT, 1), bp.reshape(1, d_in), W, asr, adr)


def _tc_norm_body(pp_ref, ss_ref, b_ref, o_ref):
    s = ss_ref[0] + ss_ref[1] + 1e-16
    o_ref[...] = (pp_ref[0] + pp_ref[1]) / s + b_ref[...]


def _tc_norm(pp, ss, b):
    d = pp.shape[-1]
    BN = 1280
    return pl.pallas_call(
        _tc_norm_body,
        grid=(N_EXT // BN,),
        in_specs=[
            pl.BlockSpec((2, BN, d), lambda i: (0, i, 0)),
            pl.BlockSpec((2, BN, 1), lambda i: (0, i, 0)),
            pl.BlockSpec((1, d), lambda i: (0, 0)),
        ],
        out_specs=pl.BlockSpec((BN, d), lambda i: (i, 0)),
        out_shape=jax.ShapeDtypeStruct((N_EXT, d), jnp.float32),
    )(pp, ss.reshape(NC, N_EXT, 1), b.reshape(1, d))


def _bcast_lane(vec, l):
    idx = jnp.full((16, 1), l, jnp.int32)
    return lax.gather(
        vec, idx,
        lax.GatherDimensionNumbers(
            offset_dims=(), collapsed_slice_dims=(0,), start_index_map=(0,)),
        slice_sizes=(1,),
        mode=lax.GatherScatterMode.PROMISE_IN_BOUNDS)

@functools.partial(jax.jit, static_argnames=("d",))
def _sc_edge(h, asv, adv, mvec, srcp, dstp, znd, zn, d):
    mesh = plsc.VectorSubcoreMesh(core_axis_name="c", subcore_axis_name="s")
    b2 = 64 if d == 128 else 128
    nch = EPW // b2
    srcp = srcp.reshape(NW, nch, b2)
    dstp = dstp.reshape(NW, nch, b2)

    @functools.partial(
        pl.kernel,
        out_type=(
            jax.ShapeDtypeStruct((NC, N_EXT, d), jnp.float32),
            jax.ShapeDtypeStruct((NC, N_EXT), jnp.float32),
        ),
        mesh=mesh,
        scratch_types=[
            pltpu.VMEM((8, b2), jnp.int32),
            pltpu.VMEM((8, b2), jnp.int32),
            [pltpu.VMEM((b2,), jnp.float32)] * 4,
            [pltpu.VMEM((b2,), jnp.float32)] * 4,
            [pltpu.VMEM((b2,), jnp.float32)] * 4,
            pltpu.VMEM((16,), jnp.float32),
            [pltpu.VMEM((b2, d), jnp.float32)] * 4,
            pltpu.VMEM_SHARED((N_EXT, d), jnp.float32),
            pltpu.VMEM_SHARED((N_EXT,), jnp.float32),
            [pltpu.SemaphoreType.DMA] * 2,
            [pltpu.SemaphoreType.DMA] * 4,
            [pltpu.SemaphoreType.DMA] * 4,
            [pltpu.SemaphoreType.DMA] * 4,
            [pltpu.SemaphoreType.DMA] * 4,
        ],
        compiler_params=pltpu.CompilerParams(use_tc_tiling_on_sc=False),
    )
    def k(h_hbm, as_hbm, ad_hbm, m_hbm, src_hbm, dst_hbm, znd_hbm, zn_hbm,
          acc_out, s_out, sidx2, didx2, ag, bg, exv, m_v, rows,
          acc_sh, s_sh, isem, asem, gsem, zsem, wsem):
        cidx = lax.axis_index("c")
        sidx = lax.axis_index("s")
        wid = sidx * NC + cidx
        r0 = sidx * RPW

        pltpu.sync_copy(znd_hbm.at[pl.ds(r0, RPW)], acc_sh.at[pl.ds(r0, RPW)])

        @pl.when(sidx == 0)
        def _():
            pltpu.sync_copy(zn_hbm, s_sh)

        pltpu.sync_copy(m_hbm, m_v)
        plsc.subcore_barrier()

        mv = m_v[...]

        def _idx_pair(chn, slot):
            sem = isem[slot % 2]
            pltpu.async_copy(
                src_hbm.at[wid, pl.ds(chn, 1)], sidx2.at[pl.ds(slot, 1)], sem)
            pltpu.async_copy(
                dst_hbm.at[wid, pl.ds(chn, 1)], didx2.at[pl.ds(slot, 1)], sem)

        def _idx_wait(slot):
            sem = isem[slot % 2]
            pltpu.make_async_copy(
                src_hbm.at[wid, pl.ds(0, 1)], sidx2.at[pl.ds(0, 1)],
                sem).wait()
            pltpu.make_async_copy(
                dst_hbm.at[wid, pl.ds(0, 1)], didx2.at[pl.ds(0, 1)],
                sem).wait()

        def _fire(slot, kb):
            pltpu.async_copy(as_hbm.at[sidx2.at[slot]], ag[kb], asem[kb])
            pltpu.async_copy(ad_hbm.at[didx2.at[slot]], bg[kb], asem[kb])
            pltpu.async_copy(h_hbm.at[sidx2.at[slot]], rows[kb], gsem[kb])

        def _wait_scalars(kb):
            pltpu.make_async_copy(as_hbm.at[sidx2.at[0]], ag[kb],
                                  asem[kb]).wait()
            pltpu.make_async_copy(ad_hbm.at[didx2.at[0]], bg[kb],
                                  asem[kb]).wait()

        def _wait_rows(kb):
            pltpu.make_async_copy(h_hbm.at[sidx2.at[0]], rows[kb],
                                  gsem[kb]).wait()

        def _drain_scatters(kb):
            pltpu.make_async_copy(exv[kb], s_sh.at[didx2.at[0]],
                                  zsem[kb]).wait()
            pltpu.make_async_copy(rows[kb], acc_sh.at[didx2.at[0]],
                                  wsem[kb]).wait()

        def _scale(buf, exb):
            for g in range(b2 // 16):
                exg = exb[pl.ds(g * 16, 16)]
                for l in range(16):
                    j = g * 16 + l
                    exj = _bcast_lane(exg, l)
                    for f in range(d // 16):
                        slf = pl.ds(f * 16, 16)
                        buf[j, slf] = buf[j, slf] * exj

        _idx_pair(0, 0)
        _idx_pair(1, 1)
        _idx_wait(0)
        _idx_wait(1)
        _fire(0, 0)
        _fire(1, 1)
        _idx_pair(2, 2)
        _idx_pair(3, 3)

        def p2(oo, carry):
            for kq in range(8):
                ch = oo * 8 + kq
                kb = kq % 4

                _wait_scalars(kb)
                for g in range(b2 // 16):
                    sl = pl.ds(g * 16, 16)
                    e = ag[kb][sl] + bg[kb][sl]
                    e = jnp.where(e > 0.0, e, e * 0.2)
                    exv[kb][sl] = jnp.exp(e - mv)

                @pl.when(ch + 2 < nch)
                def _():
                    _idx_wait(kq + 2)

                    @pl.when(ch >= 2)
                    def _():
                        _drain_scatters((kq + 2) % 4)

                    _fire((kq + 2) % 8, (kq + 2) % 4)

                @pl.when(ch + 4 < nch)
                def _():
                    _idx_pair(ch + 4, (kq + 4) % 8)

                pltpu.async_copy(exv[kb], s_sh.at[didx2.at[kq]],
                                 zsem[kb], add=True)
                _wait_rows(kb)
                _scale(rows[kb], exv[kb])
                pltpu.async_copy(rows[kb], acc_sh.at[didx2.at[kq]],
                                 wsem[kb], add=True)
            return carry

        lax.fori_loop(0, nch // 8, p2, 0)

        for kb in range(4):
            _drain_scatters(kb)
        plsc.subcore_barrier()

        pltpu.sync_copy(acc_sh.at[pl.ds(r0, RPW)],
                        acc_out.at[cidx, pl.ds(r0, RPW)])

        @pl.when(sidx == 0)
        def _():
            pltpu.sync_copy(s_sh, s_out.at[cidx])

    return k(h, asv, adv, mvec, srcp, dstp, znd, zn)


def kernel(x, edge_index, W1, a_src1, a_dst1, b1, W2, a_src2, a_dst2, b2,
           W3, a_src3, a_dst3, b3):
    src = edge_index[0].astype(jnp.int32)
    dst = edge_index[1].astype(jnp.int32)
    pad = E_PAD - E
    pad_idx = DUMMY + (jnp.arange(pad, dtype=jnp.int32) % (N_EXT - N))
    srcp = jnp.concatenate([src, pad_idx]).reshape(NW, NCH, B)
    dstp = jnp.concatenate([dst, pad_idx]).reshape(NW, NCH, B)
    x_ext = jnp.pad(x, ((0, N_EXT - N), (0, 0)))

    znd128 = jnp.zeros((N_EXT, 128), jnp.float32)
    znd64 = jnp.zeros((N_EXT, 64), jnp.float32)
    zn = jnp.zeros((N_EXT,), jnp.float32)

    h, a1, a2, m = _tc_layer(x_ext, W1, a_src1, a_dst1, first=True)
    acc, s = _sc_edge(h, a1.reshape(N_EXT), a2.reshape(N_EXT),
                      m.reshape(16), srcp, dstp, znd128, zn, d=128)

    h, a1, a2, m = _tc_layer((acc, s, b1), W2, a_src2, a_dst2, first=False)
    acc, s = _sc_edge(h, a1.reshape(N_EXT), a2.reshape(N_EXT),
                      m.reshape(16), srcp, dstp, znd64, zn, d=64)

    h, a1, a2, m = _tc_layer((acc, s, b2), W3, a_src3, a_dst3, first=False)
    acc, s = _sc_edge(h, a1.reshape(N_EXT), a2.reshape(N_EXT),
                      m.reshape(16), srcp, dstp, znd64, zn, d=64)

    out = _tc_norm(acc, s, b3)
    return out[:N]
